# baseline (speedup 1.0000x reference)
import functools

import jax
import jax.numpy as jnp
from jax import lax
from jax.experimental import pallas as pl
from jax.experimental.pallas import tpu as pltpu
from jax.experimental.pallas import tpu_sc as plsc

B, C, D = 64, 768, 576
R = B * C
NC, NS = 2, 16
NW = NC * NS
RPW = R // NW
CHUNK = 96
NCH = RPW // CHUNK


@functools.partial(
    pl.kernel,
    mesh=plsc.VectorSubcoreMesh(core_axis_name="c", subcore_axis_name="s"),
    out_type=jax.ShapeDtypeStruct((R, D), jnp.float32),
    scratch_types=[
        pltpu.VMEM((NCH, CHUNK), jnp.int32),
        pltpu.VMEM((CHUNK, D), jnp.float32),
        pltpu.VMEM((CHUNK, D), jnp.float32),
        pltpu.SemaphoreType.DMA,
        pltpu.SemaphoreType.DMA,
        pltpu.SemaphoreType.DMA,
        pltpu.SemaphoreType.DMA,
    ],
    compiler_params=pltpu.CompilerParams(use_tc_tiling_on_sc=False),
)
def _sc_permute_rows(inp_hbm, idx_hbm, out_hbm, idx_v, buf0, buf1,
                     gs0, gs1, ss0, ss1):
    wid = lax.axis_index("s") * NC + lax.axis_index("c")
    base = wid * RPW
    pltpu.sync_copy(idx_hbm.at[wid], idx_v)

    bufs = (buf0, buf1)
    gsems = (gs0, gs1)
    ssems = (ss0, ss1)
    gh = [None, None]
    sh = [None, None]

    gh[0] = pltpu.async_copy(inp_hbm.at[idx_v.at[0]], bufs[0], gsems[0])
    for c in range(NCH):
        b = c & 1
        nb = b ^ 1
        if c + 1 < NCH:
            if sh[nb] is not None:
                sh[nb].wait()
            gh[nb] = pltpu.async_copy(
                inp_hbm.at[idx_v.at[c + 1]], bufs[nb], gsems[nb])
        gh[b].wait()
        sh[b] = pltpu.async_copy(
            bufs[b], out_hbm.at[pl.ds(base + c * CHUNK, CHUNK)], ssems[b])
    sh[0].wait()
    sh[1].wait()


def kernel(inp):
    perm = jax.random.permutation(jax.random.key(1), C).astype(jnp.int32)
    idx = jnp.arange(B, dtype=jnp.int32)[:, None] * C + perm[None, :]
    idx = idx.reshape(NW, NCH, CHUNK)
    out = _sc_permute_rows(inp.reshape(R, D), idx)
    return out.reshape(B, C, D)


# speedup vs baseline: 1.3766x; 1.3766x over previous
import functools

import jax
import jax.numpy as jnp
from jax import lax
from jax.experimental import pallas as pl
from jax.experimental.pallas import tpu as pltpu
from jax.experimental.pallas import tpu_sc as plsc

B, C, D = 64, 768, 576
R = B * C
NC, NS = 2, 16
NW = NC * NS
CHUNK = 96
RPW = R // NW
NCH = RPW // CHUNK
DM = 512
DT = D - DM


@functools.partial(
    pl.kernel,
    mesh=plsc.VectorSubcoreMesh(core_axis_name="c", subcore_axis_name="s"),
    out_type=(
        jax.ShapeDtypeStruct((R, DM), jnp.float32),
        jax.ShapeDtypeStruct((R, 128), jnp.float32),
    ),
    scratch_types=[
        pltpu.VMEM((NCH, CHUNK), jnp.int32),
        pltpu.VMEM((NCH, CHUNK), jnp.int32),
        pltpu.VMEM((CHUNK, DM), jnp.float32),
        pltpu.VMEM((CHUNK, DM), jnp.float32),
        pltpu.VMEM((CHUNK, 128), jnp.float32),
        pltpu.VMEM((CHUNK, 128), jnp.float32),
        pltpu.SemaphoreType.DMA,
        pltpu.SemaphoreType.DMA,
        pltpu.SemaphoreType.DMA,
        pltpu.SemaphoreType.DMA,
    ],
)
def _sc_permute_rows(inp_hbm, tail_hbm, idx_hbm, widx_hbm,
                     main_hbm, tout_hbm,
                     idx_v, widx_v, buf0, buf1, tbuf0, tbuf1,
                     gs0, gs1, ss0, ss1):
    wid = lax.axis_index("s") * NC + lax.axis_index("c")
    pltpu.sync_copy(idx_hbm.at[wid], idx_v)
    pltpu.sync_copy(widx_hbm.at[wid], widx_v)

    bufs = (buf0, buf1)
    tbufs = (tbuf0, tbuf1)
    gsems = (gs0, gs1)
    ssems = (ss0, ss1)

    def start_gathers(c, b):
        handles = []
        for t in range(4):
            handles.append(pltpu.async_copy(
                inp_hbm.at[idx_v.at[c], pl.ds(t * 128, 128)],
                bufs[b].at[:, pl.ds(t * 128, 128)],
                gsems[b]))
        handles.append(pltpu.async_copy(
            tail_hbm.at[idx_v.at[c]], tbufs[b], gsems[b]))
        return handles

    gh = [None, None]
    sh = [None, None]

    gh[0] = start_gathers(0, 0)
    for c in range(NCH):
        b = c & 1
        nb = b ^ 1
        if c + 1 < NCH:
            if sh[nb] is not None:
                for h in sh[nb]:
                    h.wait()
            gh[nb] = start_gathers(c + 1, nb)
        for h in gh[b]:
            h.wait()
        sh[b] = [
            pltpu.async_copy(bufs[b], main_hbm.at[widx_v.at[c]], ssems[b]),
            pltpu.async_copy(tbufs[b], tout_hbm.at[widx_v.at[c]], ssems[b]),
        ]
    for hs in sh:
        for h in hs:
            h.wait()


def kernel(inp):
    perm = jax.random.permutation(jax.random.key(1), C).astype(jnp.int32)
    idx = jnp.arange(B, dtype=jnp.int32)[:, None] * C + perm[None, :]
    idx = idx.reshape(NW, NCH, CHUNK)
    widx = jnp.arange(R, dtype=jnp.int32).reshape(NW, NCH, CHUNK)
    inp2d = inp.reshape(R, D)
    tail = jnp.pad(inp2d[:, DM:], ((0, 0), (0, 128 - DT)))
    main, tout = _sc_permute_rows(inp2d, tail, idx, widx)
    out = jnp.concatenate([main, tout[:, :DT]], axis=1)
    return out.reshape(B, C, D)
